# baseline (speedup 1.0000x reference)
import jax
import jax.numpy as jnp
from jax.experimental import pallas as pl

N = 2048
OFFS = (-2, -1, 0, 1, 2)


def _mm(a, b):
    return jnp.dot(a, b, preferred_element_type=jnp.float32)


def _mm_nt(a, b):
    return jax.lax.dot_general(
        a, b, (((1,), (1,)), ((), ())), preferred_element_type=jnp.float32
    )


def _shift_rows(a, o):
    if o == 0:
        return a
    n, m = a.shape
    z = jnp.zeros((abs(o), m), dtype=a.dtype)
    if o > 0:
        return jnp.concatenate([a[o:, :], z], axis=0)
    return jnp.concatenate([z, a[:o, :]], axis=0)


def _gat_band(x, W, A, E, b, heads, amask, apply_relu):
    n = x.shape[0]
    xl = _mm(x, W)
    sa = _mm(xl, A)
    asrc = sa[:, :heads]
    adst = sa[:, heads:]
    alpha = jnp.concatenate(
        [_shift_rows(asrc, o) + adst for o in OFFS], axis=1)
    alpha = jnp.where(alpha >= 0, alpha, 0.2 * alpha) + amask
    m = alpha[:, :heads]
    for i in range(1, 5):
        m = jnp.maximum(m, alpha[:, i * heads:(i + 1) * heads])
    e = jnp.exp(alpha - jnp.concatenate([m] * 5, axis=1))
    den = e[:, :heads]
    for i in range(1, 5):
        den = den + e[:, i * heads:(i + 1) * heads]
    recip = 1.0 / (den + 1e-16)
    coef = e * jnp.concatenate([recip] * 5, axis=1)
    acc = _mm(coef[:, 2 * heads:3 * heads], E) * xl
    for i, o in enumerate(OFFS):
        if o == 0:
            continue
        acc = acc + _mm(coef[:, i * heads:(i + 1) * heads], E) * _shift_rows(xl, o)
    out = acc + b
    if apply_relu:
        out = jnp.maximum(out, 0.0)
    return out


def _layernorm(x, g, b, eps=1e-5):
    m = jnp.mean(x, axis=1, keepdims=True)
    v = jnp.mean((x - m) ** 2, axis=1, keepdims=True)
    return (x - m) * jax.lax.rsqrt(v + eps) * g + b


def _tlayer(x, Wqkv, bqkv, Wo, bo, g1, b1ln, W1, b1, W2, b2, g2, b2ln):
    n, d = x.shape
    nhead, dh = 4, d // 4
    bf = lambda a: a.astype(jnp.bfloat16)
    qkv = _mm_nt(bf(x), bf(Wqkv)) + bqkv
    q = qkv[:, :d]
    k = qkv[:, d:2 * d]
    v = qkv[:, 2 * d:]
    scale = 1.0 / jnp.sqrt(jnp.float32(dh))
    heads_out = []
    tile = 256
    for h in range(nhead):
        qh = bf(q[:, h * dh:(h + 1) * dh] * scale)
        khT = bf(k[:, h * dh:(h + 1) * dh].T)
        vh = bf(v[:, h * dh:(h + 1) * dh])
        o_tiles = []
        for t in range(n // tile):
            qt = qh[t * tile:(t + 1) * tile, :]
            s = _mm(qt, khT)
            mx = jnp.max(s, axis=1, keepdims=True)
            e = jnp.exp(s - mx)
            recip = 1.0 / jnp.sum(e, axis=1, keepdims=True)
            o_tiles.append(_mm(bf(e), vh) * recip)
        heads_out.append(jnp.concatenate(o_tiles, axis=0))
    o = jnp.concatenate(heads_out, axis=1)
    x = _layernorm(x + _mm_nt(bf(o), bf(Wo)) + bo, g1, b1ln)
    f = jnp.maximum(_mm_nt(bf(x), bf(W1)) + b1, 0.0)
    f = _mm_nt(bf(f), bf(W2)) + b2
    return _layernorm(x + f, g2, b2ln)


def _band_mask(n, heads):
    rows = jax.lax.broadcasted_iota(jnp.int32, (n, 1), 0)
    cols = []
    for o in OFFS:
        valid = (rows + o >= 0) & (rows + o < n)
        vm = jnp.where(valid, 0.0, -1e30)
        cols.append(jnp.broadcast_to(vm, (n, heads)))
    return jnp.concatenate(cols, axis=1)


def _enc_body(*refs):
    out_ref = refs[-1]
    it = iter([r[...] for r in refs[:-1]])
    x = next(it)
    mask4 = _band_mask(x.shape[0], 4)
    for _ in range(3):
        W, A, E, b = next(it), next(it), next(it), next(it)
        x = _gat_band(x, W, A, E, b, 4, mask4, True)
    out_ref[...] = x


def _t_body(*refs):
    out_ref = refs[-1]
    it = iter([r[...] for r in refs[:-1]])
    x = next(it)
    p = tuple(next(it) for _ in range(12))
    out_ref[...] = _tlayer(x, *p)


def _dec_body(*refs):
    out_ref = refs[-1]
    it = iter([r[...] for r in refs[:-1]])
    x = next(it)
    mask4 = _band_mask(x.shape[0], 4)
    mask1 = _band_mask(x.shape[0], 1)
    for _ in range(2):
        W, A, E, b = next(it), next(it), next(it), next(it)
        x = _gat_band(x, W, A, E, b, 4, mask4, True)
    W, A, E, b = next(it), next(it), next(it), next(it)
    x = _gat_band(x, W, A, E, b, 1, mask1, False)
    out_ref[...] = x.T


def kernel(window, enc0_W, enc0_att_src, enc0_att_dst, enc0_b, enc1_W, enc1_att_src, enc1_att_dst, enc1_b, enc2_W, enc2_att_src, enc2_att_dst, enc2_b, dec0_W, dec0_att_src, dec0_att_dst, dec0_b, dec1_W, dec1_att_src, dec1_att_dst, dec1_b, dec2_W, dec2_att_src, dec2_att_dst, dec2_b, t0_Wqkv, t0_bqkv, t0_Wo, t0_bo, t0_ln1_g, t0_ln1_b, t0_W1, t0_b1, t0_W2, t0_b2, t0_ln2_g, t0_ln2_b, t1_Wqkv, t1_bqkv, t1_Wo, t1_bo, t1_ln1_g, t1_ln1_b, t1_W1, t1_b1, t1_W2, t1_b2, t1_ln2_g, t1_ln2_b):
    r2 = lambda a: a.reshape(1, -1)

    def _blockdiag(att):
        heads, c = att.shape
        eye = jnp.eye(heads, dtype=att.dtype)
        return (att[:, :, None] * eye[:, None, :]).reshape(heads * c, heads)

    gat_args = []
    for (W, a_s, a_d, b) in (
        (enc0_W, enc0_att_src, enc0_att_dst, enc0_b),
        (enc1_W, enc1_att_src, enc1_att_dst, enc1_b),
        (enc2_W, enc2_att_src, enc2_att_dst, enc2_b),
        (dec0_W, dec0_att_src, dec0_att_dst, dec0_b),
        (dec1_W, dec1_att_src, dec1_att_dst, dec1_b),
        (dec2_W, dec2_att_src, dec2_att_dst, dec2_b),
    ):
        heads, c = a_s.shape
        A = jnp.concatenate([_blockdiag(a_s), _blockdiag(a_d)], axis=1)
        E = jnp.kron(jnp.eye(heads, dtype=a_s.dtype), jnp.ones((1, c), a_s.dtype))
        gat_args += [W, A, E, r2(b)]
    t_args = []
    for t in (
        (t0_Wqkv, t0_bqkv, t0_Wo, t0_bo, t0_ln1_g, t0_ln1_b,
         t0_W1, t0_b1, t0_W2, t0_b2, t0_ln2_g, t0_ln2_b),
        (t1_Wqkv, t1_bqkv, t1_Wo, t1_bo, t1_ln1_g, t1_ln1_b,
         t1_W1, t1_b1, t1_W2, t1_b2, t1_ln2_g, t1_ln2_b),
    ):
        Wqkv, bqkv, Wo, bo, g1, b1, W1, b1f, W2, b2f, g2, b2 = t
        t_args.append([Wqkv, r2(bqkv), Wo, r2(bo), r2(g1), r2(b1),
                       W1, r2(b1f), W2, r2(b2f), r2(g2), r2(b2)])

    fx32 = lambda shape: jax.ShapeDtypeStruct(shape, jnp.float32)
    x = pl.pallas_call(_enc_body, out_shape=fx32((N, 256)))(
        window, *gat_args[:12])
    x = pl.pallas_call(_t_body, out_shape=fx32((N, 256)))(x, *t_args[0])
    x = pl.pallas_call(_t_body, out_shape=fx32((N, 256)))(x, *t_args[1])
    return pl.pallas_call(_dec_body, out_shape=fx32((128, N)))(
        x, *gat_args[12:])


# speedup vs baseline: 1.3948x; 1.3948x over previous
import jax
import jax.numpy as jnp
from jax.experimental import pallas as pl

N = 2048
OFFS = (-2, -1, 0, 1, 2)


def _mm(a, b):
    return jnp.dot(a, b, preferred_element_type=jnp.float32)


def _mm_nt(a, b):
    return jax.lax.dot_general(
        a, b, (((1,), (1,)), ((), ())), preferred_element_type=jnp.float32
    )


def _shift_rows(a, o):
    if o == 0:
        return a
    n, m = a.shape
    z = jnp.zeros((abs(o), m), dtype=a.dtype)
    if o > 0:
        return jnp.concatenate([a[o:, :], z], axis=0)
    return jnp.concatenate([z, a[:o, :]], axis=0)


def _gat_band(x, W, A, E, b, heads, amasks, apply_relu):
    xl = _mm(x, W)
    sa = _mm(xl, A)
    asrc = sa[:, :heads]
    adst = sa[:, heads:]
    alphas = []
    for i, o in enumerate(OFFS):
        a = _shift_rows(asrc, o) + adst
        alphas.append(jnp.where(a >= 0, a, 0.2 * a) + amasks[i])
    m = alphas[0]
    for a in alphas[1:]:
        m = jnp.maximum(m, a)
    es = [jnp.exp(a - m) for a in alphas]
    den = es[0]
    for e in es[1:]:
        den = den + e
    recip = 1.0 / (den + 1e-16)
    acc = _mm(es[2] * recip, E) * xl
    for i, o in enumerate(OFFS):
        if o == 0:
            continue
        acc = acc + _mm(es[i] * recip, E) * _shift_rows(xl, o)
    out = acc + b
    if apply_relu:
        out = jnp.maximum(out, 0.0)
    return out


def _layernorm(x, g, b, eps=1e-5):
    m = jnp.mean(x, axis=1, keepdims=True)
    v = jnp.mean((x - m) ** 2, axis=1, keepdims=True)
    return (x - m) * jax.lax.rsqrt(v + eps) * g + b


def _tlayer(x, Wqkv, bqkv, Wo, bo, g1, b1ln, W1, b1, W2, b2, g2, b2ln):
    n, d = x.shape
    nhead, dh = 4, d // 4
    qkv = _mm_nt(x, Wqkv) + bqkv
    q = qkv[:, :d]
    k = qkv[:, d:2 * d]
    v = qkv[:, 2 * d:]
    scale = 1.0 / jnp.sqrt(jnp.float32(dh))
    heads_out = []
    tile = 256
    for h in range(nhead):
        qh = q[:, h * dh:(h + 1) * dh] * scale
        khT = k[:, h * dh:(h + 1) * dh].T
        vh = v[:, h * dh:(h + 1) * dh]
        o_tiles = []
        for t in range(n // tile):
            qt = qh[t * tile:(t + 1) * tile, :]
            s = _mm(qt, khT)
            mx = jnp.max(s, axis=1, keepdims=True)
            e = jnp.exp(s - mx)
            recip = 1.0 / jnp.sum(e, axis=1, keepdims=True)
            o_tiles.append(_mm(e, vh) * recip)
        heads_out.append(jnp.concatenate(o_tiles, axis=0))
    o = jnp.concatenate(heads_out, axis=1)
    x = _layernorm(x + _mm_nt(o, Wo) + bo, g1, b1ln)
    f = jnp.maximum(_mm_nt(x, W1) + b1, 0.0)
    f = _mm_nt(f, W2) + b2
    return _layernorm(x + f, g2, b2ln)


def _band_mask(n, heads):
    rows = jax.lax.broadcasted_iota(jnp.int32, (n, 1), 0)
    cols = []
    for o in OFFS:
        valid = (rows + o >= 0) & (rows + o < n)
        vm = jnp.where(valid, 0.0, -1e30)
        cols.append(jnp.broadcast_to(vm, (n, heads)))
    return cols


def _enc_body(*refs):
    out_ref = refs[-1]
    it = iter([r[...] for r in refs[:-1]])
    x = next(it)
    mask4 = _band_mask(x.shape[0], 4)
    for _ in range(3):
        W, A, E, b = next(it), next(it), next(it), next(it)
        x = _gat_band(x, W, A, E, b, 4, mask4, True)
    out_ref[...] = x


def _t_body(*refs):
    out_ref = refs[-1]
    it = iter([r[...] for r in refs[:-1]])
    x = next(it)
    p = tuple(next(it) for _ in range(12))
    out_ref[...] = _tlayer(x, *p)


def _dec_body(*refs):
    out_ref = refs[-1]
    it = iter([r[...] for r in refs[:-1]])
    x = next(it)
    mask4 = _band_mask(x.shape[0], 4)
    mask1 = _band_mask(x.shape[0], 1)
    for _ in range(2):
        W, A, E, b = next(it), next(it), next(it), next(it)
        x = _gat_band(x, W, A, E, b, 4, mask4, True)
    W, A, E, b = next(it), next(it), next(it), next(it)
    x = _gat_band(x, W, A, E, b, 1, mask1, False)
    out_ref[...] = x.T


def kernel(window, enc0_W, enc0_att_src, enc0_att_dst, enc0_b, enc1_W, enc1_att_src, enc1_att_dst, enc1_b, enc2_W, enc2_att_src, enc2_att_dst, enc2_b, dec0_W, dec0_att_src, dec0_att_dst, dec0_b, dec1_W, dec1_att_src, dec1_att_dst, dec1_b, dec2_W, dec2_att_src, dec2_att_dst, dec2_b, t0_Wqkv, t0_bqkv, t0_Wo, t0_bo, t0_ln1_g, t0_ln1_b, t0_W1, t0_b1, t0_W2, t0_b2, t0_ln2_g, t0_ln2_b, t1_Wqkv, t1_bqkv, t1_Wo, t1_bo, t1_ln1_g, t1_ln1_b, t1_W1, t1_b1, t1_W2, t1_b2, t1_ln2_g, t1_ln2_b):
    r2 = lambda a: a.reshape(1, -1)

    def _blockdiag(att):
        heads, c = att.shape
        eye = jnp.eye(heads, dtype=att.dtype)
        return (att[:, :, None] * eye[:, None, :]).reshape(heads * c, heads)

    gat_args = []
    for (W, a_s, a_d, b) in (
        (enc0_W, enc0_att_src, enc0_att_dst, enc0_b),
        (enc1_W, enc1_att_src, enc1_att_dst, enc1_b),
        (enc2_W, enc2_att_src, enc2_att_dst, enc2_b),
        (dec0_W, dec0_att_src, dec0_att_dst, dec0_b),
        (dec1_W, dec1_att_src, dec1_att_dst, dec1_b),
        (dec2_W, dec2_att_src, dec2_att_dst, dec2_b),
    ):
        heads, c = a_s.shape
        A = jnp.concatenate([_blockdiag(a_s), _blockdiag(a_d)], axis=1)
        E = jnp.kron(jnp.eye(heads, dtype=a_s.dtype), jnp.ones((1, c), a_s.dtype))
        gat_args += [W, A, E, r2(b)]
    t_args = []
    for t in (
        (t0_Wqkv, t0_bqkv, t0_Wo, t0_bo, t0_ln1_g, t0_ln1_b,
         t0_W1, t0_b1, t0_W2, t0_b2, t0_ln2_g, t0_ln2_b),
        (t1_Wqkv, t1_bqkv, t1_Wo, t1_bo, t1_ln1_g, t1_ln1_b,
         t1_W1, t1_b1, t1_W2, t1_b2, t1_ln2_g, t1_ln2_b),
    ):
        Wqkv, bqkv, Wo, bo, g1, b1, W1, b1f, W2, b2f, g2, b2 = t
        t_args.append([Wqkv, r2(bqkv), Wo, r2(bo), r2(g1), r2(b1),
                       W1, r2(b1f), W2, r2(b2f), r2(g2), r2(b2)])

    fx32 = lambda shape: jax.ShapeDtypeStruct(shape, jnp.float32)
    x = pl.pallas_call(_enc_body, out_shape=fx32((N, 256)))(
        window, *gat_args[:12])
    x = pl.pallas_call(_t_body, out_shape=fx32((N, 256)))(x, *t_args[0])
    x = pl.pallas_call(_t_body, out_shape=fx32((N, 256)))(x, *t_args[1])
    return pl.pallas_call(_dec_body, out_shape=fx32((128, N)))(
        x, *gat_args[12:])
